# baseline (speedup 1.0000x reference)
import functools

import jax
import jax.numpy as jnp
from jax import lax
from jax.experimental import pallas as pl
from jax.experimental.pallas import tpu as pltpu
from jax.experimental.pallas import tpu_sc as plsc

_NC = 2
_NS = 16
_NW = _NC * _NS


def _tc_copy_body(x_ref, o_ref):
    o_ref[...] = x_ref[...]


def _tc_copy(x):
    n, d = x.shape
    blk = n // 2
    spec = pl.BlockSpec((blk, d), lambda i: (i, 0))
    return pl.pallas_call(
        _tc_copy_body,
        grid=(n // blk,),
        in_specs=[spec],
        out_specs=spec,
        out_shape=jax.ShapeDtypeStruct((n, d), x.dtype),
    )(x)


def _sc_copy_body(chunk, x_hbm, o_hbm, buf, sem):
    wid = lax.axis_index("s") * _NC + lax.axis_index("c")
    base = wid * chunk
    pltpu.async_copy(x_hbm.at[pl.ds(base, chunk)], buf, sem).wait()
    pltpu.sync_copy(buf, o_hbm.at[pl.ds(base, chunk)])


def _sc_copy(x):
    n, d = x.shape
    total = n * d
    chunk = total // _NW
    flat = x.reshape(total)
    mesh = plsc.VectorSubcoreMesh(core_axis_name="c", subcore_axis_name="s")
    body = functools.partial(_sc_copy_body, chunk)
    out = pl.kernel(
        body,
        out_type=jax.ShapeDtypeStruct((total,), x.dtype),
        mesh=mesh,
        scratch_types=[
            pltpu.VMEM((chunk,), jnp.float32),
            pltpu.SemaphoreType.DMA,
        ],
    )(flat)
    return out.reshape(n, d)


def kernel(x_user, x_item, edge_index):
    del edge_index
    return (_tc_copy(x_user), _sc_copy(x_item))


# speedup vs baseline: 3.1060x; 3.1060x over previous
import jax
import jax.numpy as jnp
from jax.experimental import pallas as pl
from jax.experimental.pallas import tpu as pltpu

_SPLITS = (2000, 3000, 5000)
_OFFS = (0, 2000, 5000)


def _body(xu, xi, ou, oi, vu, vi, sin_u, sin_i, sout_u, sout_i):
    nck = len(_SPLITS)
    for k in range(nck):
        sl = pl.ds(_OFFS[k], _SPLITS[k])
        pltpu.make_async_copy(xu.at[sl], vu.at[sl], sin_u.at[k]).start()
        pltpu.make_async_copy(xi.at[sl], vi.at[sl], sin_i.at[k]).start()
    for k in range(nck):
        sl = pl.ds(_OFFS[k], _SPLITS[k])
        pltpu.make_async_copy(xu.at[sl], vu.at[sl], sin_u.at[k]).wait()
        pltpu.make_async_copy(vu.at[sl], ou.at[sl], sout_u.at[k]).start()
        pltpu.make_async_copy(xi.at[sl], vi.at[sl], sin_i.at[k]).wait()
        pltpu.make_async_copy(vi.at[sl], oi.at[sl], sout_i.at[k]).start()
    for k in range(nck):
        sl = pl.ds(_OFFS[k], _SPLITS[k])
        pltpu.make_async_copy(vu.at[sl], ou.at[sl], sout_u.at[k]).wait()
        pltpu.make_async_copy(vi.at[sl], oi.at[sl], sout_i.at[k]).wait()


def kernel(x_user, x_item, edge_index):
    del edge_index
    n, d = x_user.shape
    nck = len(_SPLITS)
    ou, oi = pl.pallas_call(
        _body,
        in_specs=[
            pl.BlockSpec(memory_space=pl.ANY),
            pl.BlockSpec(memory_space=pl.ANY),
        ],
        out_specs=[
            pl.BlockSpec(memory_space=pl.ANY),
            pl.BlockSpec(memory_space=pl.ANY),
        ],
        out_shape=[
            jax.ShapeDtypeStruct((n, d), x_user.dtype),
            jax.ShapeDtypeStruct((n, d), x_item.dtype),
        ],
        scratch_shapes=[
            pltpu.VMEM((n, d), jnp.float32),
            pltpu.VMEM((n, d), jnp.float32),
            pltpu.SemaphoreType.DMA((nck,)),
            pltpu.SemaphoreType.DMA((nck,)),
            pltpu.SemaphoreType.DMA((nck,)),
            pltpu.SemaphoreType.DMA((nck,)),
        ],
    )(x_user, x_item)
    return (ou, oi)


# speedup vs baseline: 3.2237x; 1.0379x over previous
import jax
import jax.numpy as jnp
from jax.experimental import pallas as pl
from jax.experimental.pallas import tpu as pltpu

_SPLITS = (3336, 3336, 3328)
_OFFS = (0, 3336, 6672)


def _body(xu, xi, ou, oi, vu, vi, sin_u, sin_i, sout_u, sout_i):
    nck = len(_SPLITS)
    for k in range(nck):
        sl = pl.ds(_OFFS[k], _SPLITS[k])
        pltpu.make_async_copy(xu.at[sl], vu.at[sl], sin_u.at[k]).start()
        pltpu.make_async_copy(xi.at[sl], vi.at[sl], sin_i.at[k]).start()
    for k in range(nck):
        sl = pl.ds(_OFFS[k], _SPLITS[k])
        pltpu.make_async_copy(xu.at[sl], vu.at[sl], sin_u.at[k]).wait()
        pltpu.make_async_copy(vu.at[sl], ou.at[sl], sout_u.at[k]).start()
        pltpu.make_async_copy(xi.at[sl], vi.at[sl], sin_i.at[k]).wait()
        pltpu.make_async_copy(vi.at[sl], oi.at[sl], sout_i.at[k]).start()
    for k in range(nck):
        sl = pl.ds(_OFFS[k], _SPLITS[k])
        pltpu.make_async_copy(vu.at[sl], ou.at[sl], sout_u.at[k]).wait()
        pltpu.make_async_copy(vi.at[sl], oi.at[sl], sout_i.at[k]).wait()


def kernel(x_user, x_item, edge_index):
    del edge_index
    n, d = x_user.shape
    nck = len(_SPLITS)
    ou, oi = pl.pallas_call(
        _body,
        in_specs=[
            pl.BlockSpec(memory_space=pl.ANY),
            pl.BlockSpec(memory_space=pl.ANY),
        ],
        out_specs=[
            pl.BlockSpec(memory_space=pl.ANY),
            pl.BlockSpec(memory_space=pl.ANY),
        ],
        out_shape=[
            jax.ShapeDtypeStruct((n, d), x_user.dtype),
            jax.ShapeDtypeStruct((n, d), x_item.dtype),
        ],
        scratch_shapes=[
            pltpu.VMEM((n, d), jnp.float32),
            pltpu.VMEM((n, d), jnp.float32),
            pltpu.SemaphoreType.DMA((nck,)),
            pltpu.SemaphoreType.DMA((nck,)),
            pltpu.SemaphoreType.DMA((nck,)),
            pltpu.SemaphoreType.DMA((nck,)),
        ],
    )(x_user, x_item)
    return (ou, oi)


# speedup vs baseline: 3.4724x; 1.0771x over previous
import jax
import jax.numpy as jnp
from jax.experimental import pallas as pl
from jax.experimental.pallas import tpu as pltpu

_SPLITS = (5000, 5000)
_OFFS = (0, 5000)


def _body(xu, xi, ou, oi, vu, vi, sin_u, sin_i, sout_u, sout_i):
    nck = len(_SPLITS)
    for k in range(nck):
        sl = pl.ds(_OFFS[k], _SPLITS[k])
        pltpu.make_async_copy(xu.at[sl], vu.at[sl], sin_u.at[k]).start()
    for k in range(nck):
        sl = pl.ds(_OFFS[k], _SPLITS[k])
        pltpu.make_async_copy(xi.at[sl], vi.at[sl], sin_i.at[k]).start()
    for k in range(nck):
        sl = pl.ds(_OFFS[k], _SPLITS[k])
        pltpu.make_async_copy(xu.at[sl], vu.at[sl], sin_u.at[k]).wait()
        pltpu.make_async_copy(vu.at[sl], ou.at[sl], sout_u.at[k]).start()
    for k in range(nck):
        sl = pl.ds(_OFFS[k], _SPLITS[k])
        pltpu.make_async_copy(xi.at[sl], vi.at[sl], sin_i.at[k]).wait()
        pltpu.make_async_copy(vi.at[sl], oi.at[sl], sout_i.at[k]).start()
    for k in range(nck):
        sl = pl.ds(_OFFS[k], _SPLITS[k])
        pltpu.make_async_copy(vu.at[sl], ou.at[sl], sout_u.at[k]).wait()
        pltpu.make_async_copy(vi.at[sl], oi.at[sl], sout_i.at[k]).wait()


def kernel(x_user, x_item, edge_index):
    del edge_index
    n, d = x_user.shape
    nck = len(_SPLITS)
    ou, oi = pl.pallas_call(
        _body,
        in_specs=[
            pl.BlockSpec(memory_space=pl.ANY),
            pl.BlockSpec(memory_space=pl.ANY),
        ],
        out_specs=[
            pl.BlockSpec(memory_space=pl.ANY),
            pl.BlockSpec(memory_space=pl.ANY),
        ],
        out_shape=[
            jax.ShapeDtypeStruct((n, d), x_user.dtype),
            jax.ShapeDtypeStruct((n, d), x_item.dtype),
        ],
        scratch_shapes=[
            pltpu.VMEM((n, d), jnp.float32),
            pltpu.VMEM((n, d), jnp.float32),
            pltpu.SemaphoreType.DMA((nck,)),
            pltpu.SemaphoreType.DMA((nck,)),
            pltpu.SemaphoreType.DMA((nck,)),
            pltpu.SemaphoreType.DMA((nck,)),
        ],
    )(x_user, x_item)
    return (ou, oi)
